# Initial kernel scaffold; baseline (speedup 1.0000x reference)
import jax
import jax.numpy as jnp
from jax.experimental import pallas as pl


def kernel(h, edge_index, edge_type, r, W, b):
    raise NotImplementedError("write your pallas kernel here")


# speedup vs baseline: 1.6527x; 1.6527x over previous
import functools

import jax
import jax.numpy as jnp
from jax import lax
from jax.experimental import pallas as pl
from jax.experimental.pallas import tpu as pltpu
from jax.experimental.pallas import tpu_sc as plsc

N = 10000
E = 160000
D = 128
R = 4
OUT = 128

NC = 2
NS = 16
NW = NC * NS
C = 64
NCHUNKS = E // C
BASE_TRIPS = NCHUNKS // NW
REM = NCHUNKS - BASE_TRIPS * NW
ACCW = 144
RPT = N // NS


def _tab_body(h_ref, rc_ref, o_ref):
    o_ref[0] = jnp.dot(h_ref[...], rc_ref[0], preferred_element_type=jnp.float32)


def _make_tab(h, rc):
    BM = 2000
    return pl.pallas_call(
        _tab_body,
        grid=(2 * R, N // BM),
        in_specs=[
            pl.BlockSpec((BM, D), lambda j, m: (m, 0)),
            pl.BlockSpec((1, D, D), lambda j, m: (j, 0, 0)),
        ],
        out_specs=pl.BlockSpec((1, BM, D), lambda j, m: (j, m, 0)),
        out_shape=jax.ShapeDtypeStruct((2 * R, N, D), jnp.float32),
    )(h, rc)


def _sc_body(tab, hh, srcr, dstr, typr, zer, out,
             acc, src_v, dst_v, typ_v, ia_v, ib_v, a_v, b_v, g_v, m_v, sem):
    c = lax.axis_index("c")
    s = lax.axis_index("s")
    wid = s * NC + c

    pltpu.sync_copy(zer.at[pl.ds(s * RPT, RPT)], acc.at[pl.ds(s * RPT, RPT)])

    idx16 = lax.iota(jnp.int32, 16)
    unit = jnp.where(idx16 == 0, jnp.float32(1.0), jnp.float32(0.0))

    def init_m(i, carry):
        m_v[i, pl.ds(D, 16)] = unit
        return carry

    lax.fori_loop(0, C, init_m, 0)
    plsc.subcore_barrier()

    trips = jnp.where(wid < REM, BASE_TRIPS + 1, BASE_TRIPS)

    def chunk(i, carry):
        cid = wid + NW * i
        base = pl.multiple_of(cid * C, 16)
        pltpu.sync_copy(srcr.at[pl.ds(base, C)], src_v)
        pltpu.sync_copy(dstr.at[pl.ds(base, C)], dst_v)
        pltpu.sync_copy(typr.at[pl.ds(base, C)], typ_v)
        for k in range(C // 16):
            sl = pl.ds(k * 16, 16)
            tN = typ_v[sl] * N
            ia_v[sl] = tN + dst_v[sl]
            ib_v[sl] = tN + src_v[sl] + (R * N)
        d1 = pltpu.async_copy(tab.at[ia_v], a_v, sem)
        d2 = pltpu.async_copy(tab.at[ib_v], b_v, sem)
        d3 = pltpu.async_copy(hh.at[src_v], g_v, sem)
        d1.wait()
        d2.wait()
        d3.wait()

        def edge(e, ecarry):
            for k in range(D // 16):
                sl = pl.ds(k * 16, 16)
                x = a_v[e, sl] + b_v[e, sl]
                gate = 1.0 / (1.0 + jnp.exp(-x))
                m_v[e, sl] = g_v[e, sl] * gate
            return ecarry

        lax.fori_loop(0, C, edge, 0)
        pltpu.sync_copy(m_v, acc.at[dst_v], add=True)
        return carry

    lax.fori_loop(0, trips, chunk, 0)
    plsc.subcore_barrier()
    pltpu.sync_copy(acc.at[pl.ds(s * RPT, RPT)], out.at[c, pl.ds(s * RPT, RPT)])


def _sc_call(tab, h, src, dst, typ, zer):
    mesh = plsc.VectorSubcoreMesh(
        core_axis_name="c", subcore_axis_name="s", num_cores=NC, num_subcores=NS)
    k = pl.kernel(
        _sc_body,
        out_type=jax.ShapeDtypeStruct((NC, N, ACCW), jnp.float32),
        mesh=mesh,
        compiler_params=pltpu.CompilerParams(use_tc_tiling_on_sc=False),
        scratch_types=[
            pltpu.VMEM_SHARED((N, ACCW), jnp.float32),
            pltpu.VMEM((C,), jnp.int32),
            pltpu.VMEM((C,), jnp.int32),
            pltpu.VMEM((C,), jnp.int32),
            pltpu.VMEM((C,), jnp.int32),
            pltpu.VMEM((C,), jnp.int32),
            pltpu.VMEM((C, D), jnp.float32),
            pltpu.VMEM((C, D), jnp.float32),
            pltpu.VMEM((C, D), jnp.float32),
            pltpu.VMEM((C, ACCW), jnp.float32),
            pltpu.SemaphoreType.DMA,
        ],
    )
    return k(tab, h, src, dst, typ, zer)


def _final_body(p_ref, h_ref, w_ref, b_ref, o_ref):
    ssum = p_ref[0] + p_ref[1]
    deg = ssum[:, D:D + 1]
    h_n = ssum[:, :D] / jnp.maximum(deg, 1.0)
    res = (jnp.dot(h_ref[...], w_ref[:D], preferred_element_type=jnp.float32)
           + jnp.dot(h_n, w_ref[D:], preferred_element_type=jnp.float32)
           + b_ref[...])
    o_ref[...] = jnp.where(res >= 0, res, 0.01 * res)


def _final(partials, h, W, b2):
    BM = 2000
    return pl.pallas_call(
        _final_body,
        grid=(N // BM,),
        in_specs=[
            pl.BlockSpec((NC, BM, ACCW), lambda m: (0, m, 0)),
            pl.BlockSpec((BM, D), lambda m: (m, 0)),
            pl.BlockSpec((2 * D, OUT), lambda m: (0, 0)),
            pl.BlockSpec((1, OUT), lambda m: (0, 0)),
        ],
        out_specs=pl.BlockSpec((BM, OUT), lambda m: (m, 0)),
        out_shape=jax.ShapeDtypeStruct((N, OUT), jnp.float32),
    )(partials, h, W, b2)


def kernel(h, edge_index, edge_type, r, W, b):
    rc = jnp.concatenate([r[:, :D, :], r[:, D:, :]], axis=0)
    tab = _make_tab(h, rc).reshape(2 * R * N, D)
    src = edge_index[0]
    dst = edge_index[1]
    zer = jnp.zeros((N, ACCW), jnp.float32)
    partials = _sc_call(tab, h, src, dst, edge_type, zer)
    return _final(partials, h, W, b.reshape(1, OUT))


# speedup vs baseline: 1.8555x; 1.1227x over previous
import functools

import jax
import jax.numpy as jnp
from jax import lax
from jax.experimental import pallas as pl
from jax.experimental.pallas import tpu as pltpu
from jax.experimental.pallas import tpu_sc as plsc

N = 10000
E = 160000
D = 128
R = 4
OUT = 128

NC = 2
NS = 16
NW = NC * NS
C = 32
T = -(-E // (NW * C))
E_PAD = T * NW * C
EB = E_PAD // 128
NROW = 10016
DUMMY = N
ACCW = 144
RPT = NROW // NS


def _tab_body(h_ref, rc_ref, o_ref):
    o_ref[0] = jnp.dot(h_ref[...], rc_ref[0], preferred_element_type=jnp.float32)


def _make_tab(h, rc):
    BM = 2000
    return pl.pallas_call(
        _tab_body,
        grid=(2 * R, N // BM),
        in_specs=[
            pl.BlockSpec((BM, D), lambda j, m: (m, 0)),
            pl.BlockSpec((1, D, D), lambda j, m: (j, 0, 0)),
        ],
        out_specs=pl.BlockSpec((1, BM, D), lambda j, m: (j, m, 0)),
        out_shape=jax.ShapeDtypeStruct((2 * R, N, D), jnp.float32),
    )(h, rc)


def _idx_body(s_ref, d_ref, t_ref, o_ref):
    t_n = t_ref[...] * N
    o_ref[0] = t_n + d_ref[...]
    o_ref[1] = t_n + s_ref[...] + R * N
    o_ref[2] = s_ref[...]
    o_ref[3] = d_ref[...]


def _make_idx(srcp, dstp, typp):
    return pl.pallas_call(
        _idx_body,
        out_shape=jax.ShapeDtypeStruct((4, EB, 128), jnp.int32),
    )(srcp.reshape(EB, 128), dstp.reshape(EB, 128), typp.reshape(EB, 128))


def _sc_body(tab, hh, idx4, zer, out, acc, idx_v, a_v, b_v, g_v, m_v, sem):
    c = lax.axis_index("c")
    s = lax.axis_index("s")
    wid = s * NC + c

    pltpu.sync_copy(zer.at[pl.ds(s * RPT, RPT)], acc.at[pl.ds(s * RPT, RPT)])

    idx16 = lax.iota(jnp.int32, 16)
    unit = jnp.where(idx16 == 0, jnp.float32(1.0), jnp.float32(0.0))
    for slot_ in range(2):
        def init_m(e, carry, _slot=slot_):
            m_v[_slot, e, pl.ds(D, 16)] = unit
            return carry
        lax.fori_loop(0, C, init_m, 0)
    plsc.subcore_barrier()

    def fire(i, slot):
        base = pl.multiple_of((wid + NW * i) * C, 16)
        pltpu.sync_copy(idx4.at[:, pl.ds(base, C)], idx_v.at[slot])
        pltpu.async_copy(tab.at[idx_v.at[slot, 0]], a_v.at[slot], sem.at[slot])
        pltpu.async_copy(tab.at[idx_v.at[slot, 1]], b_v.at[slot], sem.at[slot])
        pltpu.async_copy(hh.at[idx_v.at[slot, 2]], g_v.at[slot], sem.at[slot])

    fire(0, 0)

    def chunk(i, carry):
        slot = lax.rem(i, 2)
        nslot = 1 - slot

        @pl.when(i < T - 1)
        def _():
            fire(i + 1, nslot)

        pltpu.make_async_copy(tab.at[idx_v.at[slot, 0]], a_v.at[slot], sem.at[slot]).wait()
        pltpu.make_async_copy(tab.at[idx_v.at[slot, 1]], b_v.at[slot], sem.at[slot]).wait()
        pltpu.make_async_copy(hh.at[idx_v.at[slot, 2]], g_v.at[slot], sem.at[slot]).wait()

        def edge(e, ecarry):
            for k in range(D // 16):
                sl = pl.ds(k * 16, 16)
                x = a_v[slot, e, sl] + b_v[slot, e, sl]
                gate = 1.0 / (1.0 + jnp.exp(-x))
                m_v[slot, e, sl] = g_v[slot, e, sl] * gate
            return ecarry

        lax.fori_loop(0, C, edge, 0)
        pltpu.sync_copy(m_v.at[slot], acc.at[idx_v.at[slot, 3]], add=True)
        return carry

    lax.fori_loop(0, T, chunk, 0)
    plsc.subcore_barrier()
    pltpu.sync_copy(acc.at[pl.ds(s * RPT, RPT)], out.at[c, pl.ds(s * RPT, RPT)])


def _sc_call(tab, h, idx4, zer):
    mesh = plsc.VectorSubcoreMesh(
        core_axis_name="c", subcore_axis_name="s", num_cores=NC, num_subcores=NS)
    k = pl.kernel(
        _sc_body,
        out_type=jax.ShapeDtypeStruct((NC, NROW, ACCW), jnp.float32),
        mesh=mesh,
        compiler_params=pltpu.CompilerParams(use_tc_tiling_on_sc=False),
        scratch_types=[
            pltpu.VMEM_SHARED((NROW, ACCW), jnp.float32),
            pltpu.VMEM((2, 4, C), jnp.int32),
            pltpu.VMEM((2, C, D), jnp.float32),
            pltpu.VMEM((2, C, D), jnp.float32),
            pltpu.VMEM((2, C, D), jnp.float32),
            pltpu.VMEM((2, C, ACCW), jnp.float32),
            pltpu.SemaphoreType.DMA((2,)),
        ],
    )
    return k(tab, h, idx4, zer)


def _final_body(p_ref, h_ref, w_ref, b_ref, o_ref):
    ssum = p_ref[0] + p_ref[1]
    deg = ssum[:, D:D + 1]
    h_n = ssum[:, :D] / jnp.maximum(deg, 1.0)
    res = (jnp.dot(h_ref[...], w_ref[:D], preferred_element_type=jnp.float32)
           + jnp.dot(h_n, w_ref[D:], preferred_element_type=jnp.float32)
           + b_ref[...])
    o_ref[...] = jnp.where(res >= 0, res, 0.01 * res)


def _final(partials, h, W, b2):
    BM = 2000
    return pl.pallas_call(
        _final_body,
        grid=(N // BM,),
        in_specs=[
            pl.BlockSpec((NC, BM, ACCW), lambda m: (0, m, 0)),
            pl.BlockSpec((BM, D), lambda m: (m, 0)),
            pl.BlockSpec((2 * D, OUT), lambda m: (0, 0)),
            pl.BlockSpec((1, OUT), lambda m: (0, 0)),
        ],
        out_specs=pl.BlockSpec((BM, OUT), lambda m: (m, 0)),
        out_shape=jax.ShapeDtypeStruct((N, OUT), jnp.float32),
    )(partials, h, W, b2)


def kernel(h, edge_index, edge_type, r, W, b):
    rc = jnp.concatenate([r[:, :D, :], r[:, D:, :]], axis=0)
    tab = _make_tab(h, rc).reshape(2 * R * N, D)
    pad = E_PAD - E
    srcp = jnp.concatenate([edge_index[0], jnp.zeros((pad,), jnp.int32)])
    dstp = jnp.concatenate([edge_index[1], jnp.full((pad,), DUMMY, jnp.int32)])
    typp = jnp.concatenate([edge_type, jnp.zeros((pad,), jnp.int32)])
    idx4 = _make_idx(srcp, dstp, typp).reshape(4, E_PAD)
    zer = jnp.zeros((NROW, ACCW), jnp.float32)
    partials = _sc_call(tab, h, idx4, zer)
    return _final(partials, h, W, b.reshape(1, OUT))


# speedup vs baseline: 6.6624x; 3.5906x over previous
import functools

import jax
import jax.numpy as jnp
from jax import lax
from jax.experimental import pallas as pl
from jax.experimental.pallas import tpu as pltpu
from jax.experimental.pallas import tpu_sc as plsc

N = 10000
E = 160000
D = 128
R = 4
OUT = 128

NC = 2
NS = 16
NW = NC * NS
C = 32
T = -(-E // (NW * C))
E_PAD = T * NW * C
EB = E_PAD // 128
NROW = 10016
DUMMY = N
ACCW = 144
RPT = NROW // NS


def _tab_body(h_ref, rc_ref, o_ref):
    o_ref[0] = jnp.dot(h_ref[...], rc_ref[0], preferred_element_type=jnp.float32)


def _make_tab(h, rc):
    BM = 2000
    return pl.pallas_call(
        _tab_body,
        grid=(2 * R, N // BM),
        in_specs=[
            pl.BlockSpec((BM, D), lambda j, m: (m, 0)),
            pl.BlockSpec((1, D, D), lambda j, m: (j, 0, 0)),
        ],
        out_specs=pl.BlockSpec((1, BM, D), lambda j, m: (j, m, 0)),
        out_shape=jax.ShapeDtypeStruct((2 * R, N, D), jnp.float32),
    )(h, rc)


def _idx_body(s_ref, d_ref, t_ref, o_ref):
    t_n = t_ref[...] * N
    o_ref[0] = t_n + d_ref[...]
    o_ref[1] = t_n + s_ref[...] + R * N
    o_ref[2] = s_ref[...]
    o_ref[3] = d_ref[...]


def _make_idx(srcp, dstp, typp):
    return pl.pallas_call(
        _idx_body,
        out_shape=jax.ShapeDtypeStruct((4, EB, 128), jnp.int32),
    )(srcp.reshape(EB, 128), dstp.reshape(EB, 128), typp.reshape(EB, 128))


def _sc_body(tab, hh, idx4, zer, out, acc, idx_v, a_v, b_v, g_v, m_v, sem):
    c = lax.axis_index("c")
    s = lax.axis_index("s")
    wid = s * NC + c

    pltpu.sync_copy(zer.at[pl.ds(s * RPT, RPT)], acc.at[pl.ds(s * RPT, RPT)])

    idx16 = lax.iota(jnp.int32, 16)
    unit = jnp.where(idx16 == 0, jnp.float32(1.0), jnp.float32(0.0))
    for slot_ in range(2):
        def init_m(e, carry, _slot=slot_):
            m_v[_slot, e, pl.ds(D, 16)] = unit
            return carry
        lax.fori_loop(0, C, init_m, 0)
    plsc.subcore_barrier()

    def fire(i, slot):
        base = pl.multiple_of((wid + NW * i) * C, 16)
        pltpu.sync_copy(idx4.at[:, pl.ds(base, C)], idx_v.at[slot])
        pltpu.async_copy(tab.at[idx_v.at[slot, 0]], a_v.at[slot], sem.at[slot])
        pltpu.async_copy(tab.at[idx_v.at[slot, 1]], b_v.at[slot], sem.at[slot])
        pltpu.async_copy(hh.at[idx_v.at[slot, 2]], g_v.at[slot], sem.at[slot])

    fire(0, 0)

    def chunk(i, carry):
        slot = lax.rem(i, 2)
        nslot = 1 - slot

        @pl.when(i < T - 1)
        def _():
            fire(i + 1, nslot)

        pltpu.make_async_copy(tab.at[idx_v.at[slot, 0]], a_v.at[slot], sem.at[slot]).wait()
        pltpu.make_async_copy(tab.at[idx_v.at[slot, 1]], b_v.at[slot], sem.at[slot]).wait()
        pltpu.make_async_copy(hh.at[idx_v.at[slot, 2]], g_v.at[slot], sem.at[slot]).wait()

        def edge(e, ecarry):
            for k in range(D // 16):
                sl = pl.ds(k * 16, 16)
                x = a_v[slot, e, sl] + b_v[slot, e, sl]
                gate = 1.0 / (1.0 + jnp.exp(-x))
                m_v[slot, e, sl] = g_v[slot, e, sl] * gate
            return ecarry

        pltpu.sync_copy(m_v.at[slot], acc.at[idx_v.at[slot, 3]], add=True)
        return carry

    lax.fori_loop(0, T, chunk, 0)
    plsc.subcore_barrier()
    pltpu.sync_copy(acc.at[pl.ds(s * RPT, RPT)], out.at[c, pl.ds(s * RPT, RPT)])


def _sc_call(tab, h, idx4, zer):
    mesh = plsc.VectorSubcoreMesh(
        core_axis_name="c", subcore_axis_name="s", num_cores=NC, num_subcores=NS)
    k = pl.kernel(
        _sc_body,
        out_type=jax.ShapeDtypeStruct((NC, NROW, ACCW), jnp.float32),
        mesh=mesh,
        compiler_params=pltpu.CompilerParams(use_tc_tiling_on_sc=False),
        scratch_types=[
            pltpu.VMEM_SHARED((NROW, ACCW), jnp.float32),
            pltpu.VMEM((2, 4, C), jnp.int32),
            pltpu.VMEM((2, C, D), jnp.float32),
            pltpu.VMEM((2, C, D), jnp.float32),
            pltpu.VMEM((2, C, D), jnp.float32),
            pltpu.VMEM((2, C, ACCW), jnp.float32),
            pltpu.SemaphoreType.DMA((2,)),
        ],
    )
    return k(tab, h, idx4, zer)


def _final_body(p_ref, h_ref, w_ref, b_ref, o_ref):
    ssum = p_ref[0] + p_ref[1]
    deg = ssum[:, D:D + 1]
    h_n = ssum[:, :D] / jnp.maximum(deg, 1.0)
    res = (jnp.dot(h_ref[...], w_ref[:D], preferred_element_type=jnp.float32)
           + jnp.dot(h_n, w_ref[D:], preferred_element_type=jnp.float32)
           + b_ref[...])
    o_ref[...] = jnp.where(res >= 0, res, 0.01 * res)


def _final(partials, h, W, b2):
    BM = 2000
    return pl.pallas_call(
        _final_body,
        grid=(N // BM,),
        in_specs=[
            pl.BlockSpec((NC, BM, ACCW), lambda m: (0, m, 0)),
            pl.BlockSpec((BM, D), lambda m: (m, 0)),
            pl.BlockSpec((2 * D, OUT), lambda m: (0, 0)),
            pl.BlockSpec((1, OUT), lambda m: (0, 0)),
        ],
        out_specs=pl.BlockSpec((BM, OUT), lambda m: (m, 0)),
        out_shape=jax.ShapeDtypeStruct((N, OUT), jnp.float32),
    )(partials, h, W, b2)


def kernel(h, edge_index, edge_type, r, W, b):
    rc = jnp.concatenate([r[:, :D, :], r[:, D:, :]], axis=0)
    tab = _make_tab(h, rc).reshape(2 * R * N, D)
    pad = E_PAD - E
    srcp = jnp.concatenate([edge_index[0], jnp.zeros((pad,), jnp.int32)])
    dstp = jnp.concatenate([edge_index[1], jnp.full((pad,), DUMMY, jnp.int32)])
    typp = jnp.concatenate([edge_type, jnp.zeros((pad,), jnp.int32)])
    idx4 = _make_idx(srcp, dstp, typp).reshape(4, E_PAD)
    zer = jnp.zeros((NROW, ACCW), jnp.float32)
    partials = _sc_call(tab, h, idx4, zer)
    return _final(partials, h, W, b.reshape(1, OUT))


# speedup vs baseline: 6.8620x; 1.0300x over previous
import functools

import jax
import jax.numpy as jnp
from jax import lax
from jax.experimental import pallas as pl
from jax.experimental.pallas import tpu as pltpu
from jax.experimental.pallas import tpu_sc as plsc

N = 10000
E = 160000
D = 128
R = 4
OUT = 128

NC = 2
NS = 16
NW = NC * NS
C = 32
T = -(-E // (NW * C))
E_PAD = T * NW * C
EB = E_PAD // 128
NROW = 10016
DUMMY = N
DEGW = 16
RPT = NROW // NS


def _tab_body(h_ref, rc_ref, o_ref):
    o_ref[0] = jnp.dot(h_ref[...], rc_ref[0], preferred_element_type=jnp.float32)


def _make_tab(h, rc):
    BM = 2000
    return pl.pallas_call(
        _tab_body,
        grid=(2 * R, N // BM),
        in_specs=[
            pl.BlockSpec((BM, D), lambda j, m: (m, 0)),
            pl.BlockSpec((1, D, D), lambda j, m: (j, 0, 0)),
        ],
        out_specs=pl.BlockSpec((1, BM, D), lambda j, m: (j, m, 0)),
        out_shape=jax.ShapeDtypeStruct((2 * R, N, D), jnp.float32),
    )(h, rc)


def _idx_body(s_ref, d_ref, t_ref, o_ref):
    t_n = t_ref[...] * N
    o_ref[0] = t_n + d_ref[...]
    o_ref[1] = t_n + s_ref[...] + R * N
    o_ref[2] = s_ref[...]
    o_ref[3] = d_ref[...]


def _make_idx(srcp, dstp, typp):
    return pl.pallas_call(
        _idx_body,
        out_shape=jax.ShapeDtypeStruct((4, EB, 128), jnp.int32),
    )(srcp.reshape(EB, 128), dstp.reshape(EB, 128), typp.reshape(EB, 128))


def _sc_body(tab, hh, idx4, zer, out_s, out_d,
             acc, dacc, idx0, idx1, a0, a1, b0, b1, g0, g1, m0, m1, ones_v, sem):
    c = lax.axis_index("c")
    s = lax.axis_index("s")
    wid = s * NC + c

    pltpu.sync_copy(zer.at[pl.ds(s * RPT, RPT)], acc.at[pl.ds(s * RPT, RPT)])
    pltpu.sync_copy(zer.at[pl.ds(s * RPT, RPT), pl.ds(0, DEGW)],
                    dacc.at[pl.ds(s * RPT, RPT)])

    idx16 = lax.iota(jnp.int32, 16)
    unit = jnp.where(idx16 == 0, jnp.float32(1.0), jnp.float32(0.0))

    def init_ones(e, carry):
        ones_v[e, pl.ds(0, DEGW)] = unit
        return carry

    lax.fori_loop(0, C, init_ones, 0)
    plsc.subcore_barrier()

    slots = ((idx0, a0, b0, g0, m0), (idx1, a1, b1, g1, m1))

    def fire(i, sl):
        idx_v, a_v, b_v, g_v, _ = slots[sl]
        base = pl.multiple_of((wid + NW * i) * C, 16)
        pltpu.sync_copy(idx4.at[:, pl.ds(base, C)], idx_v)
        pltpu.async_copy(tab.at[idx_v.at[0]], a_v, sem.at[sl])
        pltpu.async_copy(tab.at[idx_v.at[1]], b_v, sem.at[sl])
        pltpu.async_copy(hh.at[idx_v.at[2]], g_v, sem.at[sl])

    fire(0, 0)

    def process(i, sl):
        idx_v, a_v, b_v, g_v, m_v = slots[sl]

        @pl.when(i < T - 1)
        def _():
            fire(i + 1, 1 - sl)

        pltpu.make_async_copy(tab.at[idx_v.at[0]], a_v, sem.at[sl]).wait()
        pltpu.make_async_copy(tab.at[idx_v.at[1]], b_v, sem.at[sl]).wait()
        pltpu.make_async_copy(hh.at[idx_v.at[2]], g_v, sem.at[sl]).wait()

        @functools.partial(plsc.parallel_loop, 0, C, unroll=4)
        def _edge(e):
            for k in range(D // 16):
                ds = pl.ds(k * 16, 16)
                x = a_v[e, ds] + b_v[e, ds]
                gate = 1.0 / (1.0 + jnp.exp(-x))
                m_v[e, ds] = g_v[e, ds] * gate

        pltpu.sync_copy(m_v, acc.at[idx_v.at[3]], add=True)
        pltpu.sync_copy(ones_v, dacc.at[idx_v.at[3]], add=True)

    def chunk(i, carry):
        lax.cond(lax.rem(i, 2) == 0,
                 lambda: process(i, 0),
                 lambda: process(i, 1))
        return carry

    lax.fori_loop(0, T, chunk, 0)
    plsc.subcore_barrier()
    pltpu.sync_copy(acc.at[pl.ds(s * RPT, RPT)], out_s.at[c, pl.ds(s * RPT, RPT)])
    pltpu.sync_copy(dacc.at[pl.ds(s * RPT, RPT)], out_d.at[c, pl.ds(s * RPT, RPT)])


def _sc_call(tab, h, idx4, zer):
    mesh = plsc.VectorSubcoreMesh(
        core_axis_name="c", subcore_axis_name="s", num_cores=NC, num_subcores=NS)
    k = pl.kernel(
        _sc_body,
        out_type=(jax.ShapeDtypeStruct((NC, NROW, D), jnp.float32),
                  jax.ShapeDtypeStruct((NC, NROW, DEGW), jnp.float32)),
        mesh=mesh,
        compiler_params=pltpu.CompilerParams(use_tc_tiling_on_sc=False),
        scratch_types=[
            pltpu.VMEM_SHARED((NROW, D), jnp.float32),
            pltpu.VMEM_SHARED((NROW, DEGW), jnp.float32),
            pltpu.VMEM((4, C), jnp.int32),
            pltpu.VMEM((4, C), jnp.int32),
            pltpu.VMEM((C, D), jnp.float32),
            pltpu.VMEM((C, D), jnp.float32),
            pltpu.VMEM((C, D), jnp.float32),
            pltpu.VMEM((C, D), jnp.float32),
            pltpu.VMEM((C, D), jnp.float32),
            pltpu.VMEM((C, D), jnp.float32),
            pltpu.VMEM((C, D), jnp.float32),
            pltpu.VMEM((C, D), jnp.float32),
            pltpu.VMEM((C, DEGW), jnp.float32),
            pltpu.SemaphoreType.DMA((2,)),
        ],
    )
    return k(tab, h, idx4, zer)


def _final_body(p_ref, d_ref, h_ref, w_ref, b_ref, o_ref):
    ssum = p_ref[0] + p_ref[1]
    deg = d_ref[0, :, :1] + d_ref[1, :, :1]
    h_n = ssum / jnp.maximum(deg, 1.0)
    res = (jnp.dot(h_ref[...], w_ref[:D], preferred_element_type=jnp.float32)
           + jnp.dot(h_n, w_ref[D:], preferred_element_type=jnp.float32)
           + b_ref[...])
    o_ref[...] = jnp.where(res >= 0, res, 0.01 * res)


def _final(psum, pdeg, h, W, b2):
    BM = 2000
    return pl.pallas_call(
        _final_body,
        grid=(N // BM,),
        in_specs=[
            pl.BlockSpec((NC, BM, D), lambda m: (0, m, 0)),
            pl.BlockSpec((NC, BM, DEGW), lambda m: (0, m, 0)),
            pl.BlockSpec((BM, D), lambda m: (m, 0)),
            pl.BlockSpec((2 * D, OUT), lambda m: (0, 0)),
            pl.BlockSpec((1, OUT), lambda m: (0, 0)),
        ],
        out_specs=pl.BlockSpec((BM, OUT), lambda m: (m, 0)),
        out_shape=jax.ShapeDtypeStruct((N, OUT), jnp.float32),
    )(psum, pdeg, h, W, b2)


def kernel(h, edge_index, edge_type, r, W, b):
    rc = jnp.concatenate([r[:, :D, :], r[:, D:, :]], axis=0)
    tab = _make_tab(h, rc).reshape(2 * R * N, D)
    pad = E_PAD - E
    srcp = jnp.concatenate([edge_index[0], jnp.zeros((pad,), jnp.int32)])
    dstp = jnp.concatenate([edge_index[1], jnp.full((pad,), DUMMY, jnp.int32)])
    typp = jnp.concatenate([edge_type, jnp.zeros((pad,), jnp.int32)])
    idx4 = _make_idx(srcp, dstp, typp).reshape(4, E_PAD)
    zer = jnp.zeros((NROW, D), jnp.float32)
    psum, pdeg = _sc_call(tab, h, idx4, zer)
    return _final(psum, pdeg, h, W, b.reshape(1, OUT))
